# baseline (speedup 1.0000x reference)
import jax
import jax.numpy as jnp
from jax.experimental import pallas as pl

BETA = 0.25
TILE = 256
_F32 = jnp.float32


def _rqvae_tile(x_ref, ew0, eb0, ew1, eb1, ew2, eb2,
                dw0, db0, dw1, db1, dw2, db2,
                cbt0, cbt1, cbt2, out_ref):
    x = x_ref[...]
    h = jnp.maximum(jnp.dot(x, ew0[...], preferred_element_type=_F32) + eb0[0, :], 0.0)
    h = jnp.maximum(jnp.dot(h, ew1[...], preferred_element_type=_F32) + eb1[0, :], 0.0)
    res0 = jnp.dot(h, ew2[...], preferred_element_type=_F32) + eb2[0, :]

    res = res0
    acc = jnp.zeros((x.shape[0],), _F32)
    for cbt_ref in (cbt0, cbt1, cbt2):
        cbt = cbt_ref[...]
        k = cbt.shape[1]
        cb2 = jnp.sum(cbt * cbt, axis=0)
        dist = cb2[None, :] - 2.0 * jnp.dot(res, cbt, preferred_element_type=_F32)
        minv = jnp.min(dist, axis=-1, keepdims=True)
        iota = jax.lax.broadcasted_iota(jnp.int32, dist.shape, 1)
        idx = jnp.min(jnp.where(dist == minv, iota, k), axis=-1, keepdims=True)
        onehot = (iota == idx).astype(_F32)
        emb = jax.lax.dot_general(onehot, cbt, (((1,), (1,)), ((), ())),
                                  preferred_element_type=_F32)
        res = res - emb
        acc = acc + jnp.sum(res * res, axis=-1)

    e = res0 - res
    h = jnp.maximum(jnp.dot(e, dw0[...], preferred_element_type=_F32) + db0[0, :], 0.0)
    h = jnp.maximum(jnp.dot(h, dw1[...], preferred_element_type=_F32) + db1[0, :], 0.0)
    x_hat = jnp.dot(h, dw2[...], preferred_element_type=_F32) + db2[0, :]
    d = x_hat - x
    recon = jnp.sum(d * d, axis=-1)
    partial = jnp.sum(recon + (1.0 + BETA) * acc)
    out_ref[...] = jnp.full((1, 1, 128), partial, _F32)


def kernel(x, gumbel_t, enc_w0, enc_b0, enc_w1, enc_b1, enc_w2, enc_b2,
           dec_w0, dec_b0, dec_w1, dec_b1, dec_w2, dec_b2,
           codebook0, codebook1, codebook2):
    b = x.shape[0]
    num_tiles = b // TILE
    biases = [jnp.reshape(v, (1, -1)) for v in
              (enc_b0, enc_b1, enc_b2, dec_b0, dec_b1, dec_b2)]
    cbts = [codebook0.T, codebook1.T, codebook2.T]

    def whole(a):
        return pl.BlockSpec(a.shape, lambda i: (0,) * a.ndim)

    ops = [enc_w0, biases[0], enc_w1, biases[1], enc_w2, biases[2],
           dec_w0, biases[3], dec_w1, biases[4], dec_w2, biases[5]] + cbts
    in_specs = [pl.BlockSpec((TILE, x.shape[1]), lambda i: (i, 0))]
    in_specs += [whole(a) for a in ops]

    partials = pl.pallas_call(
        _rqvae_tile,
        grid=(num_tiles,),
        in_specs=in_specs,
        out_specs=pl.BlockSpec((1, 1, 128), lambda i: (i, 0, 0)),
        out_shape=jax.ShapeDtypeStruct((num_tiles, 1, 128), _F32),
    )(x, *ops)
    return jnp.sum(partials[:, 0, 0]) / b
